# baseline (speedup 1.0000x reference)
import functools

import jax
import jax.numpy as jnp
from jax.experimental import pallas as pl

D_IN = 4096
H1 = 200
B = 16
L = 1024

JT = 1024
DT = 1024


def _hist_body(idx_ref, h_ref):
    jt = pl.program_id(0)
    bins = jt * JT + jax.lax.broadcasted_iota(jnp.int32, (1, 1, JT), 2)
    chunk = 128

    def body(c, acc):
        seg = idx_ref[:, pl.ds(c * chunk, chunk)]
        eq = (seg[:, :, None] == bins).astype(jnp.float32)
        return acc + jnp.sum(eq, axis=1)

    h_ref[...] = jax.lax.fori_loop(0, L // chunk, body,
                                   jnp.zeros((B, JT), jnp.float32))


def _attn_mlp_body(h_ref, w1_ref, a_ref, wsa_ref, bsa_ref,
                   w2_ref, b2_ref, w3_ref, b3_ref, w4_ref, b4_ref, pre_ref):
    f32 = jnp.float32
    w1 = w1_ref[...]
    h = h_ref[...]
    s = jax.lax.dot(a_ref[...], w1, preferred_element_type=f32)
    e = jnp.exp(jnp.tanh(s))
    denom = jax.lax.dot_general(h, e, (((1,), (1,)), ((), ())),
                                preferred_element_type=f32)
    r = wsa_ref[...] / denom
    w = jax.lax.dot(r, e, preferred_element_type=f32)
    f = h * w
    lz = jax.lax.dot_general(f, w1, (((1,), (1,)), ((), ())),
                             preferred_element_type=f32) + bsa_ref[0, 0]
    z = jnp.tanh(lz)
    z2 = jnp.tanh(jax.lax.dot_general(z, w2_ref[...], (((1,), (1,)), ((), ())),
                                      preferred_element_type=f32) + b2_ref[...])
    dz = jnp.tanh(jax.lax.dot_general(z2, w3_ref[...], (((1,), (1,)), ((), ())),
                                      preferred_element_type=f32) + b3_ref[...])
    pre_ref[...] = jax.lax.dot_general(dz, w4_ref[...], (((1,), (1,)), ((), ())),
                                       preferred_element_type=f32) + b4_ref[...]


NSPLIT = 4
QT = JT // NSPLIT


def _main_body(w1t_ref, w4t_ref, pc0, pc1, pc2, pc3, h_ref, pre_ref, out_ref):
    f32 = jnp.float32
    jt = pl.program_id(1)
    njt = pl.num_programs(1)
    part = jnp.zeros((B, DT), f32)
    for q, pcq in enumerate((pc0, pc1, pc2, pc3)):
        w1tq = w1t_ref[pl.ds(jt * JT + q * QT, QT), :]
        m = jax.lax.dot(w1tq, w4t_ref[...], preferred_element_type=f32)
        c = (m * pcq[...]).astype(jnp.bfloat16)
        hq = h_ref[:, pl.ds(jt * JT + q * QT, QT)].astype(jnp.bfloat16)
        part = part + jax.lax.dot(hq, c, preferred_element_type=f32)

    @pl.when(jt == 0)
    def _():
        out_ref[...] = part

    @pl.when(jt > 0)
    def _():
        out_ref[...] += part

    @pl.when(jt == njt - 1)
    def _():
        out_ref[...] = jax.nn.sigmoid(out_ref[...] + pre_ref[...])


@functools.partial(jax.jit, static_argnames=("interpret",))
def kernel(batch_item_index, place_correlation, W1, W2, b2, W3, b3, W4, b4,
           A, Wsa, bsa, interpret=False):
    f32 = jnp.float32
    h = pl.pallas_call(
        _hist_body,
        grid=(D_IN // JT,),
        in_specs=[pl.BlockSpec((B, L), lambda j: (0, 0))],
        out_specs=pl.BlockSpec((B, JT), lambda j: (0, j)),
        out_shape=jax.ShapeDtypeStruct((B, D_IN), f32),
        interpret=interpret,
    )(batch_item_index)

    pre = pl.pallas_call(
        _attn_mlp_body,
        out_shape=jax.ShapeDtypeStruct((B, D_IN), f32),
        interpret=interpret,
    )(h, W1, A, Wsa, bsa.reshape(1, 1), W2, b2.reshape(1, -1),
      W3, b3.reshape(1, -1), W4, b4.reshape(1, -1))

    w1t = W1.T.astype(jnp.bfloat16)
    w4t = W4.T.astype(jnp.bfloat16)
    y = pl.pallas_call(
        _main_body,
        grid=(D_IN // DT, D_IN // JT),
        in_specs=[
            pl.BlockSpec((D_IN, H1), lambda d, j: (0, 0)),
            pl.BlockSpec((H1, DT), lambda d, j: (0, d)),
        ] + [
            pl.BlockSpec((QT, DT), lambda d, j, q=q: (NSPLIT * j + q, d))
            for q in range(NSPLIT)
        ] + [
            pl.BlockSpec((B, D_IN), lambda d, j: (0, 0)),
            pl.BlockSpec((B, DT), lambda d, j: (0, d)),
        ],
        out_specs=pl.BlockSpec((B, DT), lambda d, j: (0, d)),
        out_shape=jax.ShapeDtypeStruct((B, D_IN), f32),
        interpret=interpret,
    )(w1t, w4t, place_correlation, place_correlation,
      place_correlation, place_correlation, h, pre)
    return y


# speedup vs baseline: 1.0400x; 1.0400x over previous
import functools

import jax
import jax.numpy as jnp
from jax.experimental import pallas as pl
from jax.experimental.pallas import tpu as pltpu

D_IN = 4096
H1 = 200
B = 16
L = 1024

JT = 1024
DT = 1024


def _hist_body(idx_ref, h_ref):
    jt = pl.program_id(0)
    bins = jt * JT + jax.lax.broadcasted_iota(jnp.int32, (1, 1, JT), 2)
    chunk = 128

    def body(c, acc):
        seg = idx_ref[:, pl.ds(c * chunk, chunk)]
        eq = (seg[:, :, None] == bins).astype(jnp.float32)
        return acc + jnp.sum(eq, axis=1)

    h_ref[...] = jax.lax.fori_loop(0, L // chunk, body,
                                   jnp.zeros((B, JT), jnp.float32))


def _attn_mlp_body(h_ref, w1_ref, a_ref, wsa_ref, bsa_ref,
                   w2_ref, b2_ref, w3_ref, b3_ref, w4_ref, b4_ref, pre_ref):
    f32 = jnp.float32
    w1 = w1_ref[...]
    h = h_ref[...]
    s = jax.lax.dot(a_ref[...], w1, preferred_element_type=f32)
    e = jnp.exp(jnp.tanh(s))
    denom = jax.lax.dot_general(h, e, (((1,), (1,)), ((), ())),
                                preferred_element_type=f32)
    r = wsa_ref[...] / denom
    w = jax.lax.dot(r, e, preferred_element_type=f32)
    f = h * w
    lz = jax.lax.dot_general(f, w1, (((1,), (1,)), ((), ())),
                             preferred_element_type=f32) + bsa_ref[0, 0]
    z = jnp.tanh(lz)
    z2 = jnp.tanh(jax.lax.dot_general(z, w2_ref[...], (((1,), (1,)), ((), ())),
                                      preferred_element_type=f32) + b2_ref[...])
    dz = jnp.tanh(jax.lax.dot_general(z2, w3_ref[...], (((1,), (1,)), ((), ())),
                                      preferred_element_type=f32) + b3_ref[...])
    pre_ref[...] = jax.lax.dot_general(dz, w4_ref[...], (((1,), (1,)), ((), ())),
                                       preferred_element_type=f32) + b4_ref[...]


NSPLIT = 4
QT = JT // NSPLIT


def _main_body(w1t_ref, w4t_ref, pc_ref, h_ref, pre_ref, out_ref):
    f32 = jnp.float32
    jt = pl.program_id(1)
    njt = pl.num_programs(1)
    w1tj = w1t_ref[pl.ds(jt * JT, JT), :]
    m = jax.lax.dot(w1tj, w4t_ref[...], preferred_element_type=f32)
    c = (m * pc_ref[...]).astype(jnp.bfloat16)
    hj = h_ref[:, pl.ds(jt * JT, JT)].astype(jnp.bfloat16)
    part = jax.lax.dot(hj, c, preferred_element_type=f32)

    @pl.when(jt == 0)
    def _():
        out_ref[...] = part

    @pl.when(jt > 0)
    def _():
        out_ref[...] += part

    @pl.when(jt == njt - 1)
    def _():
        out_ref[...] = jax.nn.sigmoid(out_ref[...] + pre_ref[...])


@functools.partial(jax.jit, static_argnames=("interpret",))
def kernel(batch_item_index, place_correlation, W1, W2, b2, W3, b3, W4, b4,
           A, Wsa, bsa, interpret=False):
    f32 = jnp.float32
    h = pl.pallas_call(
        _hist_body,
        grid=(D_IN // JT,),
        in_specs=[pl.BlockSpec((B, L), lambda j: (0, 0))],
        out_specs=pl.BlockSpec((B, JT), lambda j: (0, j)),
        out_shape=jax.ShapeDtypeStruct((B, D_IN), f32),
        interpret=interpret,
    )(batch_item_index)

    pre = pl.pallas_call(
        _attn_mlp_body,
        out_shape=jax.ShapeDtypeStruct((B, D_IN), f32),
        interpret=interpret,
    )(h, W1, A, Wsa, bsa.reshape(1, 1), W2, b2.reshape(1, -1),
      W3, b3.reshape(1, -1), W4, b4.reshape(1, -1))

    w1t = W1.T.astype(jnp.bfloat16)
    w4t = W4.T.astype(jnp.bfloat16)
    y = pl.pallas_call(
        _main_body,
        grid=(D_IN // DT, D_IN // JT),
        in_specs=[
            pl.BlockSpec((D_IN, H1), lambda d, j: (0, 0)),
            pl.BlockSpec((H1, DT), lambda d, j: (0, d)),
            pl.BlockSpec((JT, DT), lambda d, j: (j, d)),
            pl.BlockSpec((B, D_IN), lambda d, j: (0, 0)),
            pl.BlockSpec((B, DT), lambda d, j: (0, d)),
        ],
        out_specs=pl.BlockSpec((B, DT), lambda d, j: (0, d)),
        out_shape=jax.ShapeDtypeStruct((B, D_IN), f32),
        compiler_params=pltpu.CompilerParams(
            dimension_semantics=("parallel", "arbitrary")),
        interpret=interpret,
    )(w1t, w4t, place_correlation, h, pre)
    return y


# speedup vs baseline: 1.0689x; 1.0278x over previous
import functools

import jax
import jax.numpy as jnp
from jax.experimental import pallas as pl
from jax.experimental.pallas import tpu as pltpu

D_IN = 4096
H1 = 200
B = 16
L = 1024

JT = 512


def _hist_body(idx_ref, h_ref):
    jt = pl.program_id(0)
    bins = jt * JT + jax.lax.broadcasted_iota(jnp.int32, (1, 1, JT), 2)
    chunk = 128

    def body(c, acc):
        seg = idx_ref[:, pl.ds(c * chunk, chunk)]
        eq = (seg[:, :, None] == bins).astype(jnp.float32)
        return acc + jnp.sum(eq, axis=1)

    h_ref[...] = jax.lax.fori_loop(0, L // chunk, body,
                                   jnp.zeros((B, JT), jnp.float32))


def _attn_mlp_body(h_ref, w1_ref, a_ref, wsa_ref, bsa_ref,
                   w2_ref, b2_ref, w3_ref, b3_ref, w4_ref, b4_ref, pre_ref):
    f32 = jnp.float32
    w1 = w1_ref[...]
    h = h_ref[...]
    s = jax.lax.dot(a_ref[...], w1, preferred_element_type=f32)
    e = jnp.exp(jnp.tanh(s))
    denom = jax.lax.dot_general(h, e, (((1,), (1,)), ((), ())),
                                preferred_element_type=f32)
    r = wsa_ref[...] / denom
    w = jax.lax.dot(r, e, preferred_element_type=f32)
    f = h * w
    lz = jax.lax.dot_general(f, w1, (((1,), (1,)), ((), ())),
                             preferred_element_type=f32) + bsa_ref[0, 0]
    z = jnp.tanh(lz)
    z2 = jnp.tanh(jax.lax.dot_general(z, w2_ref[...], (((1,), (1,)), ((), ())),
                                      preferred_element_type=f32) + b2_ref[...])
    dz = jnp.tanh(jax.lax.dot_general(z2, w3_ref[...], (((1,), (1,)), ((), ())),
                                      preferred_element_type=f32) + b3_ref[...])
    pre_ref[...] = jax.lax.dot_general(dz, w4_ref[...], (((1,), (1,)), ((), ())),
                                       preferred_element_type=f32) + b4_ref[...]


NSPLIT = 4
QT = JT // NSPLIT


def _main_body(w1t_ref, w4t_ref, pc_ref, h_ref, pre_ref, out_ref):
    f32 = jnp.float32
    jt = pl.program_id(0)
    njt = pl.num_programs(0)
    w1tj = w1t_ref[...]
    m = jax.lax.dot(w1tj, w4t_ref[...], preferred_element_type=f32)
    c = (m * pc_ref[...]).astype(jnp.bfloat16)
    hj = h_ref[:, pl.ds(jt * JT, JT)].astype(jnp.bfloat16)
    part = jax.lax.dot(hj, c, preferred_element_type=f32)

    @pl.when(jt == 0)
    def _():
        out_ref[...] = part

    @pl.when(jt > 0)
    def _():
        out_ref[...] += part

    @pl.when(jt == njt - 1)
    def _():
        out_ref[...] = jax.nn.sigmoid(out_ref[...] + pre_ref[...])


@functools.partial(jax.jit, static_argnames=("interpret",))
def kernel(batch_item_index, place_correlation, W1, W2, b2, W3, b3, W4, b4,
           A, Wsa, bsa, interpret=False):
    f32 = jnp.float32
    h = pl.pallas_call(
        _hist_body,
        grid=(D_IN // JT,),
        in_specs=[pl.BlockSpec((B, L), lambda j: (0, 0))],
        out_specs=pl.BlockSpec((B, JT), lambda j: (0, j)),
        out_shape=jax.ShapeDtypeStruct((B, D_IN), f32),
        interpret=interpret,
    )(batch_item_index)

    pre = pl.pallas_call(
        _attn_mlp_body,
        out_shape=jax.ShapeDtypeStruct((B, D_IN), f32),
        interpret=interpret,
    )(h, W1, A, Wsa, bsa.reshape(1, 1), W2, b2.reshape(1, -1),
      W3, b3.reshape(1, -1), W4, b4.reshape(1, -1))

    w1t = W1.T.astype(jnp.bfloat16)
    w4t = W4.T.astype(jnp.bfloat16)
    y = pl.pallas_call(
        _main_body,
        grid=(D_IN // JT,),
        in_specs=[
            pl.BlockSpec((JT, H1), lambda j: (j, 0)),
            pl.BlockSpec((H1, D_IN), lambda j: (0, 0)),
            pl.BlockSpec((JT, D_IN), lambda j: (j, 0)),
            pl.BlockSpec((B, D_IN), lambda j: (0, 0)),
            pl.BlockSpec((B, D_IN), lambda j: (0, 0)),
        ],
        out_specs=pl.BlockSpec((B, D_IN), lambda j: (0, 0)),
        out_shape=jax.ShapeDtypeStruct((B, D_IN), f32),
        interpret=interpret,
    )(w1t, w4t, place_correlation, h, pre)
    return y
